# baseline (speedup 1.0000x reference)
import jax
import jax.numpy as jnp
from jax import lax
from jax.experimental import pallas as pl
from jax.experimental.pallas import tpu as pltpu
from jax.experimental.pallas import tpu_sc as plsc

_DIM = 1024
_SCALE = _DIM ** (-0.5)
_NC = 2
_NS = 16
_NW = _NC * _NS
_F = 8192
_SLOTS = 6


def _sc_body(emb_hbm, out_hbm, *scratch):
    in_bufs = scratch[:_SLOTS]
    out_bufs = scratch[_SLOTS:2 * _SLOTS]
    load_sems = scratch[2 * _SLOTS:3 * _SLOTS]
    store_sems = scratch[3 * _SLOTS:4 * _SLOTS]

    total = emb_hbm.shape[0]
    per_w = total // _NW
    n = per_w // _F

    wid = lax.axis_index("s") * _NC + lax.axis_index("c")
    base = wid * per_w

    def load(i):
        b = i % _SLOTS
        return pltpu.async_copy(
            emb_hbm.at[pl.ds(pl.multiple_of(base + i * _F, 8), _F)],
            in_bufs[b], load_sems[b])

    def store(i):
        b = i % _SLOTS
        return pltpu.async_copy(
            out_bufs[b],
            out_hbm.at[pl.ds(pl.multiple_of(base + i * _F, 8), _F)],
            store_sems[b])

    loads = {i: load(i) for i in range(min(_SLOTS, n))}
    stores = {}
    for i in range(n):
        b = i % _SLOTS
        loads.pop(i).wait()
        if i >= _SLOTS:
            stores.pop(i - _SLOTS).wait()

        ib, ob = in_bufs[b], out_bufs[b]

        @plsc.parallel_loop(0, _F, 16, unroll=16)
        def _scale(j):
            ob[pl.ds(j, 16)] = ib[pl.ds(j, 16)] * _SCALE

        stores[i] = store(i)
        if i + _SLOTS < n:
            loads[i + _SLOTS] = load(i + _SLOTS)
    for i in sorted(stores):
        stores.pop(i).wait()


def kernel(x, emb):
    rows, dim = emb.shape
    total = rows * dim
    assert total % (_NW * _F) == 0
    mesh = plsc.VectorSubcoreMesh(
        core_axis_name="c", subcore_axis_name="s",
        num_cores=_NC, num_subcores=_NS)
    scratch = (
        [pltpu.VMEM((_F,), jnp.float32) for _ in range(2 * _SLOTS)]
        + [pltpu.SemaphoreType.DMA for _ in range(2 * _SLOTS)]
    )
    out_flat = pl.kernel(
        _sc_body,
        out_type=jax.ShapeDtypeStruct((total,), emb.dtype),
        mesh=mesh,
        scratch_types=scratch,
    )(emb.reshape(total))
    return out_flat.reshape(rows, dim)


# speedup vs baseline: 1.0090x; 1.0090x over previous
import jax
import jax.numpy as jnp
from jax import lax
from jax.experimental import pallas as pl
from jax.experimental.pallas import tpu as pltpu
from jax.experimental.pallas import tpu_sc as plsc

_DIM = 1024
_SCALE = _DIM ** (-0.5)
_NC = 2
_NS = 16
_NW = _NC * _NS
_F = 16384
_SLOTS = 3


def _sc_body(emb_hbm, out_hbm, *scratch):
    in_bufs = scratch[:_SLOTS]
    out_bufs = scratch[_SLOTS:2 * _SLOTS]
    load_sems = scratch[2 * _SLOTS:3 * _SLOTS]
    store_sems = scratch[3 * _SLOTS:4 * _SLOTS]

    total = emb_hbm.shape[0]
    per_w = total // _NW
    n = per_w // _F

    wid = lax.axis_index("s") * _NC + lax.axis_index("c")
    base = wid * per_w

    def load(i):
        b = i % _SLOTS
        return pltpu.async_copy(
            emb_hbm.at[pl.ds(pl.multiple_of(base + i * _F, 8), _F)],
            in_bufs[b], load_sems[b])

    def store(i):
        b = i % _SLOTS
        return pltpu.async_copy(
            out_bufs[b],
            out_hbm.at[pl.ds(pl.multiple_of(base + i * _F, 8), _F)],
            store_sems[b])

    loads = {i: load(i) for i in range(min(_SLOTS, n))}
    stores = {}
    for i in range(n):
        b = i % _SLOTS
        loads.pop(i).wait()
        if i >= _SLOTS:
            stores.pop(i - _SLOTS).wait()

        ib, ob = in_bufs[b], out_bufs[b]

        @plsc.parallel_loop(0, _F, 16, unroll=16)
        def _scale(j):
            ob[pl.ds(j, 16)] = ib[pl.ds(j, 16)] * _SCALE

        stores[i] = store(i)
        if i + _SLOTS < n:
            loads[i + _SLOTS] = load(i + _SLOTS)
    for i in sorted(stores):
        stores.pop(i).wait()


def kernel(x, emb):
    rows, dim = emb.shape
    total = rows * dim
    assert total % (_NW * _F) == 0
    mesh = plsc.VectorSubcoreMesh(
        core_axis_name="c", subcore_axis_name="s",
        num_cores=_NC, num_subcores=_NS)
    scratch = (
        [pltpu.VMEM((_F,), jnp.float32) for _ in range(2 * _SLOTS)]
        + [pltpu.SemaphoreType.DMA for _ in range(2 * _SLOTS)]
    )
    out_flat = pl.kernel(
        _sc_body,
        out_type=jax.ShapeDtypeStruct((total,), emb.dtype),
        mesh=mesh,
        scratch_types=scratch,
    )(emb.reshape(total))
    return out_flat.reshape(rows, dim)


# speedup vs baseline: 5.2213x; 5.1746x over previous
import jax
import jax.numpy as jnp
from jax.experimental import pallas as pl

_DIM = 1024
_SCALE = _DIM ** (-0.5)
_BLOCK_ROWS = 2048


def _scale_kernel(emb_ref, out_ref):
    out_ref[...] = emb_ref[...] * _SCALE


def kernel(x, emb):
    seq_len = x.shape[1]
    rows = emb.shape[0]
    assert seq_len == rows
    grid = rows // _BLOCK_ROWS
    return pl.pallas_call(
        _scale_kernel,
        grid=(grid,),
        in_specs=[pl.BlockSpec((_BLOCK_ROWS, _DIM), lambda i: (i, 0))],
        out_specs=pl.BlockSpec((_BLOCK_ROWS, _DIM), lambda i: (i, 0)),
        out_shape=jax.ShapeDtypeStruct((rows, _DIM), emb.dtype),
    )(emb)


# speedup vs baseline: 5.2345x; 1.0025x over previous
import jax
import jax.numpy as jnp
from jax.experimental import pallas as pl
from jax.experimental.pallas import tpu as pltpu

_DIM = 1024
_SCALE = _DIM ** (-0.5)
_CHUNKS = (256, 256, 512, 1024, 2048, 2048, 1024, 512, 256, 256)
_MAX_ROWS = max(_CHUNKS)
_K = 3


def _stream_scale_kernel(emb_hbm, out_hbm, in_slots, out_slots, load_sems,
                         store_sems):
    offs = []
    o = 0
    for r in _CHUNKS:
        offs.append(o)
        o += r
    n = len(_CHUNKS)

    def load(i):
        s = i % _K
        pltpu.make_async_copy(
            emb_hbm.at[pl.ds(offs[i], _CHUNKS[i]), :],
            in_slots.at[s, pl.ds(0, _CHUNKS[i]), :], load_sems.at[s]).start()

    def store_copy(i):
        s = i % _K
        return pltpu.make_async_copy(
            out_slots.at[s, pl.ds(0, _CHUNKS[i]), :],
            out_hbm.at[pl.ds(offs[i], _CHUNKS[i]), :], store_sems.at[s])

    for i in range(min(_K, n)):
        load(i)
    for i in range(n):
        s = i % _K
        pltpu.make_async_copy(
            emb_hbm.at[pl.ds(offs[i], _CHUNKS[i]), :],
            in_slots.at[s, pl.ds(0, _CHUNKS[i]), :], load_sems.at[s]).wait()
        if i >= _K:
            store_copy(i - _K).wait()
        out_slots[s, pl.ds(0, _CHUNKS[i]), :] = (
            in_slots[s, pl.ds(0, _CHUNKS[i]), :] * _SCALE)
        store_copy(i).start()
        if i + _K < n:
            load(i + _K)
    for i in range(max(0, n - _K), n):
        store_copy(i).wait()


def kernel(x, emb):
    seq_len = x.shape[1]
    rows, dim = emb.shape
    assert seq_len == rows and dim == _DIM and sum(_CHUNKS) == rows
    return pl.pallas_call(
        _stream_scale_kernel,
        in_specs=[pl.BlockSpec(memory_space=pl.ANY)],
        out_specs=pl.BlockSpec(memory_space=pl.ANY),
        out_shape=jax.ShapeDtypeStruct((rows, dim), emb.dtype),
        scratch_shapes=[
            pltpu.VMEM((_K, _MAX_ROWS, _DIM), jnp.float32),
            pltpu.VMEM((_K, _MAX_ROWS, _DIM), jnp.float32),
            pltpu.SemaphoreType.DMA((_K,)),
            pltpu.SemaphoreType.DMA((_K,)),
        ],
    )(emb)


# speedup vs baseline: 5.3022x; 1.0129x over previous
import jax
import jax.numpy as jnp
from jax.experimental import pallas as pl
from jax.experimental.pallas import tpu as pltpu

_DIM = 1024
_SCALE = _DIM ** (-0.5)
_CHUNKS = (256, 1792, 2048, 2048, 1792, 256)
_MAX_ROWS = max(_CHUNKS)
_K = 3


def _stream_scale_kernel(emb_hbm, out_hbm, in_slots, out_slots, load_sems,
                         store_sems):
    offs = []
    o = 0
    for r in _CHUNKS:
        offs.append(o)
        o += r
    n = len(_CHUNKS)

    def load(i):
        s = i % _K
        pltpu.make_async_copy(
            emb_hbm.at[pl.ds(offs[i], _CHUNKS[i]), :],
            in_slots.at[s, pl.ds(0, _CHUNKS[i]), :], load_sems.at[s]).start()

    def store_copy(i):
        s = i % _K
        return pltpu.make_async_copy(
            out_slots.at[s, pl.ds(0, _CHUNKS[i]), :],
            out_hbm.at[pl.ds(offs[i], _CHUNKS[i]), :], store_sems.at[s])

    for i in range(min(_K, n)):
        load(i)
    for i in range(n):
        s = i % _K
        pltpu.make_async_copy(
            emb_hbm.at[pl.ds(offs[i], _CHUNKS[i]), :],
            in_slots.at[s, pl.ds(0, _CHUNKS[i]), :], load_sems.at[s]).wait()
        if i >= _K:
            store_copy(i - _K).wait()
        out_slots[s, pl.ds(0, _CHUNKS[i]), :] = (
            in_slots[s, pl.ds(0, _CHUNKS[i]), :] * _SCALE)
        store_copy(i).start()
        if i + _K < n:
            load(i + _K)
    for i in range(max(0, n - _K), n):
        store_copy(i).wait()


def kernel(x, emb):
    seq_len = x.shape[1]
    rows, dim = emb.shape
    assert seq_len == rows and dim == _DIM and sum(_CHUNKS) == rows
    return pl.pallas_call(
        _stream_scale_kernel,
        in_specs=[pl.BlockSpec(memory_space=pl.ANY)],
        out_specs=pl.BlockSpec(memory_space=pl.ANY),
        out_shape=jax.ShapeDtypeStruct((rows, dim), emb.dtype),
        scratch_shapes=[
            pltpu.VMEM((_K, _MAX_ROWS, _DIM), jnp.float32),
            pltpu.VMEM((_K, _MAX_ROWS, _DIM), jnp.float32),
            pltpu.SemaphoreType.DMA((_K,)),
            pltpu.SemaphoreType.DMA((_K,)),
        ],
    )(emb)
